# baseline (speedup 1.0000x reference)
import jax
import jax.numpy as jnp
from jax import lax
from jax.experimental import pallas as pl
from jax.experimental.pallas import tpu as pltpu
from jax.experimental.pallas import tpu_sc as plsc

VOCAB = 100000
D_MODEL = 1024
SCALE = 32.0

NUM_CORES = 2
NUM_SUBCORES = 16
NUM_WORKERS = NUM_CORES * NUM_SUBCORES
LANES = 16

N_TOKENS = 4 * 4096
TOK_PER_WORKER = N_TOKENS // NUM_WORKERS
CHUNK = 32
N_CHUNKS = TOK_PER_WORKER // CHUNK
NBUF = 3
VECS_PER_ROW = D_MODEL // LANES
VECS_PER_CHUNK = CHUNK * VECS_PER_ROW


def _body(ids_hbm, table_hbm, out_hbm, idx_v, *rest):
  bufs = rest[:NBUF]
  gsem = rest[NBUF : 2 * NBUF]
  ssem = rest[2 * NBUF :]

  c = lax.axis_index("c")
  s = lax.axis_index("s")
  wid = s * NUM_CORES + c
  row = wid // 8
  tbase = (wid % 8) * TOK_PER_WORKER

  pltpu.sync_copy(ids_hbm.at[row, pl.ds(tbase, TOK_PER_WORKER)], idx_v)

  def gather(h, b):
    pltpu.make_async_copy(
        table_hbm.at[idx_v.at[pl.ds(h * CHUNK, CHUNK)]], bufs[b], gsem[b]
    ).start()

  def wait_gather(b):
    pltpu.make_async_copy(
        table_hbm.at[idx_v.at[pl.ds(0, CHUNK)]], bufs[b], gsem[b]
    ).wait()

  def scatter(h, b):
    pltpu.make_async_copy(
        bufs[b], out_hbm.at[row, pl.ds(tbase + h * CHUNK, CHUNK)], ssem[b]
    ).start()

  def wait_scatter(b):
    pltpu.make_async_copy(
        bufs[b], out_hbm.at[row, pl.ds(tbase, CHUNK)], ssem[b]
    ).wait()

  def scale(b):
    ref = bufs[b]

    @plsc.parallel_loop(0, VECS_PER_CHUNK, 1, unroll=8)
    def _vec(v):
      r = lax.shift_right_logical(v, 6)
      col = lax.shift_left(lax.bitwise_and(v, VECS_PER_ROW - 1), 4)
      sl = pl.ds(pl.multiple_of(col, LANES), LANES)
      ref[r, sl] = ref[r, sl] * SCALE

  gather(0, 0)

  for h in range(2):
    gather(h + 1, h + 1)
    wait_gather(h)
    scale(h)
    scatter(h, h)

  def outer(o, carry):
    h0 = 2 + o * NBUF
    for k in range(NBUF):
      h = h0 + k
      b = (2 + k) % NBUF
      b1 = k % NBUF
      wait_scatter(b1)
      gather(h + 1, b1)
      wait_gather(b)
      scale(b)
      scatter(h, b)
    return carry

  lax.fori_loop(0, (N_CHUNKS - 4) // NBUF, outer, 0)

  wait_scatter(0)
  gather(N_CHUNKS - 1, 0)
  wait_gather(2)
  scale(2)
  scatter(N_CHUNKS - 2, 2)
  wait_gather(0)
  scale(0)
  scatter(N_CHUNKS - 1, 0)
  for b in range(NBUF):
    wait_scatter(b)


@jax.jit
def _embed(ids, table):
  mesh = plsc.VectorSubcoreMesh(core_axis_name="c", subcore_axis_name="s")
  return pl.kernel(
      _body,
      out_type=jax.ShapeDtypeStruct((4, 4096, D_MODEL), jnp.float32),
      mesh=mesh,
      scratch_types=[pltpu.VMEM((TOK_PER_WORKER,), jnp.int32)]
      + [pltpu.VMEM((CHUNK, D_MODEL), jnp.float32) for _ in range(NBUF)]
      + [pltpu.SemaphoreType.DMA for _ in range(2 * NBUF)],
  )(ids, table)


def kernel(token_ids, embedding):
  return _embed(token_ids.astype(jnp.int32), embedding)


# speedup vs baseline: 1.0187x; 1.0187x over previous
import jax
import jax.numpy as jnp
from jax import lax
from jax.experimental import pallas as pl
from jax.experimental.pallas import tpu as pltpu
from jax.experimental.pallas import tpu_sc as plsc

VOCAB = 100000
D_MODEL = 1024
SCALE = 32.0

NUM_CORES = 2
NUM_SUBCORES = 16
NUM_WORKERS = NUM_CORES * NUM_SUBCORES
LANES = 16

N_TOKENS = 4 * 4096
TOK_PER_WORKER = N_TOKENS // NUM_WORKERS
CHUNK = 16
N_CHUNKS = TOK_PER_WORKER // CHUNK
DEPTH = 2
NBUF = 2 * DEPTH
VECS_PER_ROW = D_MODEL // LANES
VECS_PER_CHUNK = CHUNK * VECS_PER_ROW


def _body(ids_hbm, table_hbm, out_hbm, idx_v, *rest):
  bufs = rest[:NBUF]
  gsem = rest[NBUF : 2 * NBUF]
  ssem = rest[2 * NBUF :]

  c = lax.axis_index("c")
  s = lax.axis_index("s")
  wid = s * NUM_CORES + c
  row = wid // 8
  tbase = (wid % 8) * TOK_PER_WORKER

  pltpu.sync_copy(ids_hbm.at[row, pl.ds(tbase, TOK_PER_WORKER)], idx_v)

  def gather(h, b):
    pltpu.make_async_copy(
        table_hbm.at[idx_v.at[pl.ds(h * CHUNK, CHUNK)]], bufs[b], gsem[b]
    ).start()

  def wait_gather(b):
    pltpu.make_async_copy(
        table_hbm.at[idx_v.at[pl.ds(0, CHUNK)]], bufs[b], gsem[b]
    ).wait()

  def scatter(h, b):
    pltpu.make_async_copy(
        bufs[b], out_hbm.at[row, pl.ds(tbase + h * CHUNK, CHUNK)], ssem[b]
    ).start()

  def wait_scatter(b):
    pltpu.make_async_copy(
        bufs[b], out_hbm.at[row, pl.ds(tbase, CHUNK)], ssem[b]
    ).wait()

  def scale(b):
    ref = bufs[b]

    @plsc.parallel_loop(0, VECS_PER_CHUNK, 1, unroll=16)
    def _vec(v):
      r = lax.shift_right_logical(v, 6)
      col = lax.shift_left(lax.bitwise_and(v, VECS_PER_ROW - 1), 4)
      sl = pl.ds(pl.multiple_of(col, LANES), LANES)
      ref[r, sl] = ref[r, sl] * SCALE

  for h in range(DEPTH):
    gather(h, h)
  for h in range(DEPTH):
    wait_gather(h)
    gather(h + DEPTH, h + DEPTH)
    scale(h)
    scatter(h, h)

  n_main = (N_CHUNKS - 2 * DEPTH) // NBUF * NBUF

  def outer(o, carry):
    h0 = DEPTH + o * NBUF
    for k in range(NBUF):
      h = h0 + k
      b = (DEPTH + k) % NBUF
      b2 = k % NBUF
      wait_scatter(b2)
      gather(h + DEPTH, b2)
      wait_gather(b)
      scale(b)
      scatter(h, b)
    return carry

  lax.fori_loop(0, n_main // NBUF, outer, 0)

  for h in range(DEPTH + n_main, N_CHUNKS):
    b = h % NBUF
    wait_gather(b)
    if h + DEPTH < N_CHUNKS:
      b2 = (h + DEPTH) % NBUF
      wait_scatter(b2)
      gather(h + DEPTH, b2)
    scale(b)
    scatter(h, b)
  for h in range(N_CHUNKS - NBUF, N_CHUNKS):
    wait_scatter(h % NBUF)


@jax.jit
def _embed(ids, table):
  mesh = plsc.VectorSubcoreMesh(core_axis_name="c", subcore_axis_name="s")
  return pl.kernel(
      _body,
      out_type=jax.ShapeDtypeStruct((4, 4096, D_MODEL), jnp.float32),
      mesh=mesh,
      scratch_types=[pltpu.VMEM((TOK_PER_WORKER,), jnp.int32)]
      + [pltpu.VMEM((CHUNK, D_MODEL), jnp.float32) for _ in range(NBUF)]
      + [pltpu.SemaphoreType.DMA for _ in range(2 * NBUF)],
  )(ids, table)


def kernel(token_ids, embedding):
  return _embed(token_ids.astype(jnp.int32), embedding)


# speedup vs baseline: 1.0211x; 1.0023x over previous
import jax
import jax.numpy as jnp
from jax import lax
from jax.experimental import pallas as pl
from jax.experimental.pallas import tpu as pltpu
from jax.experimental.pallas import tpu_sc as plsc

VOCAB = 100000
D_MODEL = 1024
SCALE = 32.0

NUM_CORES = 2
NUM_SUBCORES = 16
NUM_WORKERS = NUM_CORES * NUM_SUBCORES
LANES = 16

N_TOKENS = 4 * 4096
TOK_PER_WORKER = N_TOKENS // NUM_WORKERS
CHUNK = 16
N_CHUNKS = TOK_PER_WORKER // CHUNK
DEPTH = 2
NBUF = 2 * DEPTH
VECS_PER_ROW = D_MODEL // LANES
VECS_PER_CHUNK = CHUNK * VECS_PER_ROW


def _body(ids_hbm, table_hbm, out_hbm, idx_v, *rest):
  bufs = rest[:NBUF]
  gsem = rest[NBUF : 2 * NBUF]
  ssem = rest[2 * NBUF :]

  c = lax.axis_index("c")
  s = lax.axis_index("s")
  wid = s * NUM_CORES + c
  row = wid // 8
  tbase = (wid % 8) * TOK_PER_WORKER

  pltpu.sync_copy(ids_hbm.at[row, pl.ds(tbase, TOK_PER_WORKER)], idx_v)

  def gather(h, b):
    pltpu.make_async_copy(
        table_hbm.at[idx_v.at[pl.ds(h * CHUNK, CHUNK)]], bufs[b], gsem[b]
    ).start()

  def wait_gather(b):
    pltpu.make_async_copy(
        table_hbm.at[idx_v.at[pl.ds(0, CHUNK)]], bufs[b], gsem[b]
    ).wait()

  def scatter(h, b):
    pltpu.make_async_copy(
        bufs[b], out_hbm.at[row, pl.ds(tbase + h * CHUNK, CHUNK)], ssem[b]
    ).start()

  def wait_scatter(b):
    pltpu.make_async_copy(
        bufs[b], out_hbm.at[row, pl.ds(tbase, CHUNK)], ssem[b]
    ).wait()

  def scale(b):
    ref = bufs[b]

    @plsc.parallel_loop(0, VECS_PER_CHUNK, 1, unroll=8)
    def _vec(v):
      r = lax.shift_right_logical(v, 6)
      col = lax.shift_left(lax.bitwise_and(v, VECS_PER_ROW - 1), 4)
      sl = pl.ds(pl.multiple_of(col, LANES), LANES)
      ref[r, sl] = ref[r, sl] * SCALE

  for h in range(DEPTH):
    gather(h, h)
  for h in range(DEPTH):
    wait_gather(h)
    gather(h + DEPTH, h + DEPTH)
    scale(h)
    scatter(h, h)

  n_main = (N_CHUNKS - 2 * DEPTH) // NBUF * NBUF

  def outer(o, carry):
    h0 = DEPTH + o * NBUF
    for k in range(NBUF):
      h = h0 + k
      b = (DEPTH + k) % NBUF
      b2 = k % NBUF
      wait_scatter(b2)
      gather(h + DEPTH, b2)
      wait_gather(b)
      scale(b)
      scatter(h, b)
    return carry

  lax.fori_loop(0, n_main // NBUF, outer, 0)

  for h in range(DEPTH + n_main, N_CHUNKS):
    b = h % NBUF
    wait_gather(b)
    if h + DEPTH < N_CHUNKS:
      b2 = (h + DEPTH) % NBUF
      wait_scatter(b2)
      gather(h + DEPTH, b2)
    scale(b)
    scatter(h, b)
  for h in range(N_CHUNKS - NBUF, N_CHUNKS):
    wait_scatter(h % NBUF)


@jax.jit
def _embed(ids, table):
  mesh = plsc.VectorSubcoreMesh(core_axis_name="c", subcore_axis_name="s")
  return pl.kernel(
      _body,
      out_type=jax.ShapeDtypeStruct((4, 4096, D_MODEL), jnp.float32),
      mesh=mesh,
      scratch_types=[pltpu.VMEM((TOK_PER_WORKER,), jnp.int32)]
      + [pltpu.VMEM((CHUNK, D_MODEL), jnp.float32) for _ in range(NBUF)]
      + [pltpu.SemaphoreType.DMA for _ in range(2 * NBUF)],
  )(ids, table)


def kernel(token_ids, embedding):
  return _embed(token_ids.astype(jnp.int32), embedding)
